# Initial kernel scaffold; baseline (speedup 1.0000x reference)
import jax
import jax.numpy as jnp
from jax import lax
from jax.experimental import pallas as pl
from jax.experimental.pallas import tpu as pltpu
from jax.experimental.pallas import tpu_sc as plsc

_B = 4096
_L = 200
_D = 50
_OUT = 20

_NC = 2
_NS = 16
_NW = _NC * _NS
_SPT = _B // _NW

_CH0 = 128
_CH1 = _L - _CH0


def _embed_sums(x, lengths, table):
    mesh = plsc.VectorSubcoreMesh(core_axis_name="c", subcore_axis_name="s")

    def body(x_hbm, len_hbm, table_hbm, sums_hbm, xv, lv, acc, rows0, rows1,
             sem0, sem1):
        wid = lax.axis_index("s") * _NC + lax.axis_index("c")
        base = wid * _SPT
        pltpu.sync_copy(x_hbm.at[pl.ds(base, _SPT)], xv)
        pltpu.sync_copy(len_hbm.at[pl.ds(base, _SPT)], lv)

        rows = (rows0, rows1)
        sems = (sem0, sem1)

        def fire(s, buf, sem):
            pltpu.async_copy(table_hbm.at[xv.at[s, pl.ds(0, _CH0)]],
                             buf.at[pl.ds(0, _CH0)], sem)
            pltpu.async_copy(table_hbm.at[xv.at[s, pl.ds(_CH0, _CH1)]],
                             buf.at[pl.ds(_CH0, _CH1)], sem)

        def drain(buf, sem):
            pltpu.make_async_copy(table_hbm.at[pl.ds(0, _L)], buf, sem).wait()

        fire(0, rows0, sem0)

        @pl.loop(0, _SPT, step=2)
        def _(s0):
            for bsel in range(2):
                s = s0 + bsel
                buf = rows[bsel]
                sem = sems[bsel]

                @pl.when(s + 1 < _SPT)
                def _():
                    fire(s + 1, rows[1 - bsel], sems[1 - bsel])

                drain(buf, sem)

                n = lv[s]
                z = jnp.zeros((16,), jnp.float32)

                def accum(j, carry):
                    a0, a1, a2, a3 = carry
                    return (a0 + buf[j, pl.ds(0, 16)],
                            a1 + buf[j, pl.ds(16, 16)],
                            a2 + buf[j, pl.ds(32, 16)],
                            a3 + buf[j, pl.ds(34, 16)])

                a0, a1, a2, a3 = lax.fori_loop(0, n, accum, (z, z, z, z))
                acc[s, pl.ds(0, 16)] = a0
                acc[s, pl.ds(16, 16)] = a1
                acc[s, pl.ds(32, 16)] = a2
                acc[s, pl.ds(34, 16)] = a3

        pltpu.sync_copy(acc, sums_hbm.at[pl.ds(base, _SPT)])

    run = pl.kernel(
        body,
        out_type=jax.ShapeDtypeStruct((_B, _D), jnp.float32),
        mesh=mesh,
        scratch_types=[
            pltpu.VMEM((_SPT, _L), jnp.int32),
            pltpu.VMEM((_SPT,), jnp.int32),
            pltpu.VMEM((_SPT, _D), jnp.float32),
            pltpu.VMEM((_L, _D), jnp.float32),
            pltpu.VMEM((_L, _D), jnp.float32),
            pltpu.SemaphoreType.DMA,
            pltpu.SemaphoreType.DMA,
        ],
    )
    return run(x, lengths, table)


def _head_body(sums_ref, len_ref, w_ref, b_ref, out_ref):
    s = sums_ref[...]
    l = len_ref[...].astype(jnp.float32)
    rep = jnp.maximum(s / l, 0.0)
    out_ref[...] = lax.dot_general(
        rep, w_ref[...], (((1,), (1,)), ((), ())),
        preferred_element_type=jnp.float32) + b_ref[...]


def _head(sums, lengths, W, b):
    return pl.pallas_call(
        _head_body,
        out_shape=jax.ShapeDtypeStruct((_B, _OUT), jnp.float32),
    )(sums, lengths, W, b)


def kernel(x, lengths, table, W, b):
    xi = x.astype(jnp.int32)
    li = lengths.astype(jnp.int32)
    sums = _embed_sums(xi, li, table)
    return _head(sums, li.reshape(_B, 1), W, b.reshape(1, _OUT))


# speedup vs baseline: 8.4800x; 8.4800x over previous
import jax
import jax.numpy as jnp
from jax import lax
from jax.experimental import pallas as pl
from jax.experimental.pallas import tpu as pltpu
from jax.experimental.pallas import tpu_sc as plsc

_B = 4096
_L = 200
_D = 50
_DP = 56
_OUT = 20

_NC = 2
_NS = 16
_NW = _NC * _NS
_SPT = _B // _NW

_CH0 = 128
_CH1 = _L - _CH0


def _embed_sums(x, lengths, table):
    mesh = plsc.VectorSubcoreMesh(core_axis_name="c", subcore_axis_name="s")

    def body(x_hbm, len_hbm, table_hbm, sums_hbm, xv, lv, acc, buf, sem0):
        wid = lax.axis_index("s") * _NC + lax.axis_index("c")
        base = wid * _SPT
        pltpu.sync_copy(x_hbm.at[pl.ds(base, _SPT)], xv)
        pltpu.sync_copy(len_hbm.at[pl.ds(base, _SPT)], lv.at[pl.ds(0, _SPT)])

        @pl.loop(0, _SPT)
        def _(s):
            cp1 = pltpu.async_copy(table_hbm.at[xv.at[s, pl.ds(0, _CH0)]],
                                   buf.at[pl.ds(0, _CH0)], sem0)
            cp2 = pltpu.async_copy(table_hbm.at[xv.at[s, pl.ds(_CH0, _CH1)]],
                                   buf.at[pl.ds(_CH0, _CH1)], sem0)
            cp1.wait()
            cp2.wait()

            n = lv[pl.ds(s, 16)][0]
            z = jnp.zeros((16,), jnp.float32)

            def accum(j, carry):
                a0, a1, a2, a3 = carry
                return (a0 + buf[j, pl.ds(0, 16)],
                        a1 + buf[j, pl.ds(16, 16)],
                        a2 + buf[j, pl.ds(32, 16)],
                        a3 + buf[j, pl.ds(40, 16)])

            a0, a1, a2, a3 = lax.fori_loop(0, n, accum, (z, z, z, z))
            acc[s, pl.ds(0, 16)] = a0
            acc[s, pl.ds(16, 16)] = a1
            acc[s, pl.ds(32, 16)] = a2
            acc[s, pl.ds(40, 16)] = a3

        pltpu.sync_copy(acc, sums_hbm.at[pl.ds(base, _SPT)])

    run = pl.kernel(
        body,
        out_type=jax.ShapeDtypeStruct((_B, _DP), jnp.float32),
        mesh=mesh,
        scratch_types=[
            pltpu.VMEM((_SPT, _L), jnp.int32),
            pltpu.VMEM((_SPT + 16,), jnp.int32),
            pltpu.VMEM((_SPT, _DP), jnp.float32),
            pltpu.VMEM((_L, _DP), jnp.float32),
            pltpu.SemaphoreType.DMA,
        ],
        compiler_params=pltpu.CompilerParams(use_tc_tiling_on_sc=False),
    )
    return run(x, lengths, table)


def _head_body(sums_ref, len_ref, w_ref, b_ref, out_ref):
    s = sums_ref[:, :_D]
    l = len_ref[...].astype(jnp.float32)
    rep = jnp.maximum(s / l, 0.0)
    out_ref[...] = lax.dot_general(
        rep, w_ref[...], (((1,), (1,)), ((), ())),
        preferred_element_type=jnp.float32) + b_ref[...]


def _head(sums, lengths, W, b):
    return pl.pallas_call(
        _head_body,
        out_shape=jax.ShapeDtypeStruct((_B, _OUT), jnp.float32),
    )(sums, lengths, W, b)


def kernel(x, lengths, table, W, b):
    xi = x.astype(jnp.int32)
    li = lengths.astype(jnp.int32)
    tp = jnp.pad(table, ((0, 0), (0, _DP - _D)))
    sums = _embed_sums(xi, li, tp)
    return _head(sums, li.reshape(_B, 1), W, b.reshape(1, _OUT))


# speedup vs baseline: 12.6471x; 1.4914x over previous
import jax
import jax.numpy as jnp
from jax import lax
from jax.experimental import pallas as pl
from jax.experimental.pallas import tpu as pltpu
from jax.experimental.pallas import tpu_sc as plsc

_B = 4096
_L = 200
_D = 50
_DP = 56
_OUT = 20

_NC = 2
_NS = 16
_NW = _NC * _NS
_SPT = _B // _NW

_CH = 32
_NCH = (_L + _CH - 1) // _CH
_LP = _NCH * _CH


def _embed_sums(x, lengths, table):
    mesh = plsc.VectorSubcoreMesh(core_axis_name="c", subcore_axis_name="s")

    def body(x_hbm, len_hbm, table_hbm, sums_hbm, xv, lv, acc, buf0, buf1,
             sem0, sem1):
        wid = lax.axis_index("s") * _NC + lax.axis_index("c")
        base = wid * _SPT
        pltpu.sync_copy(x_hbm.at[pl.ds(base, _SPT)], xv)
        pltpu.sync_copy(len_hbm.at[pl.ds(base, _SPT)], lv.at[pl.ds(0, _SPT)])

        def nchunks(s):
            n = lv[pl.ds(s, 16)][0]
            return (n + (_CH - 1)) // _CH

        def fire(s, buf, sem):
            m = nchunks(s)

            @pl.loop(0, m)
            def _(c):
                pltpu.async_copy(
                    table_hbm.at[xv.at[s, pl.ds(c * _CH, _CH)]],
                    buf.at[pl.ds(c * _CH, _CH)], sem)

        def drain(s, buf, sem):
            m = nchunks(s)

            @pl.loop(0, m)
            def _(c):
                pltpu.make_async_copy(
                    table_hbm.at[xv.at[s, pl.ds(c * _CH, _CH)]],
                    buf.at[pl.ds(c * _CH, _CH)], sem).wait()

        def process(s, buf):
            n = lv[pl.ds(s, 16)][0]
            z = jnp.zeros((16,), jnp.float32)
            n4 = (n // 4) * 4

            def accum4(j, carry):
                a0, a1, a2, a3, b0, b1, b2, b3 = carry
                a0 += buf[j, pl.ds(0, 16)]
                a1 += buf[j, pl.ds(16, 16)]
                a2 += buf[j, pl.ds(32, 16)]
                a3 += buf[j, pl.ds(40, 16)]
                b0 += buf[j + 1, pl.ds(0, 16)]
                b1 += buf[j + 1, pl.ds(16, 16)]
                b2 += buf[j + 1, pl.ds(32, 16)]
                b3 += buf[j + 1, pl.ds(40, 16)]
                a0 += buf[j + 2, pl.ds(0, 16)]
                a1 += buf[j + 2, pl.ds(16, 16)]
                a2 += buf[j + 2, pl.ds(32, 16)]
                a3 += buf[j + 2, pl.ds(40, 16)]
                b0 += buf[j + 3, pl.ds(0, 16)]
                b1 += buf[j + 3, pl.ds(16, 16)]
                b2 += buf[j + 3, pl.ds(32, 16)]
                b3 += buf[j + 3, pl.ds(40, 16)]
                return a0, a1, a2, a3, b0, b1, b2, b3

            def accum1(j, carry):
                a0, a1, a2, a3, b0, b1, b2, b3 = carry
                a0 += buf[j, pl.ds(0, 16)]
                a1 += buf[j, pl.ds(16, 16)]
                a2 += buf[j, pl.ds(32, 16)]
                a3 += buf[j, pl.ds(40, 16)]
                return a0, a1, a2, a3, b0, b1, b2, b3

            carry = (z, z, z, z, z, z, z, z)
            carry = pl.loop(0, n4, step=4, init_carry=carry)(accum4)
            carry = pl.loop(n4, n, init_carry=carry)(accum1)
            a0, a1, a2, a3, b0, b1, b2, b3 = carry
            acc[s, pl.ds(0, 16)] = a0 + b0
            acc[s, pl.ds(16, 16)] = a1 + b1
            acc[s, pl.ds(32, 16)] = a2 + b2
            acc[s, pl.ds(40, 16)] = a3 + b3

        fire(0, buf0, sem0)

        @pl.loop(0, _SPT, step=2)
        def _(s0):
            fire(s0 + 1, buf1, sem1)
            drain(s0, buf0, sem0)
            process(s0, buf0)
            s2 = jnp.where(s0 + 2 >= _SPT, 0, s0 + 2)
            fire(s2, buf0, sem0)
            drain(s0 + 1, buf1, sem1)
            process(s0 + 1, buf1)

        drain(0, buf0, sem0)
        pltpu.sync_copy(acc, sums_hbm.at[pl.ds(base, _SPT)])

    run = pl.kernel(
        body,
        out_type=jax.ShapeDtypeStruct((_B, _DP), jnp.float32),
        mesh=mesh,
        scratch_types=[
            pltpu.VMEM((_SPT, _L), jnp.int32),
            pltpu.VMEM((_SPT + 16,), jnp.int32),
            pltpu.VMEM((_SPT, _DP), jnp.float32),
            pltpu.VMEM((_LP, _DP), jnp.float32),
            pltpu.VMEM((_LP, _DP), jnp.float32),
            pltpu.SemaphoreType.DMA,
            pltpu.SemaphoreType.DMA,
        ],
        compiler_params=pltpu.CompilerParams(use_tc_tiling_on_sc=False),
    )
    return run(x, lengths, table)


def _head_body(sums_ref, len_ref, w_ref, b_ref, out_ref):
    s = sums_ref[:, :_D]
    l = len_ref[...].astype(jnp.float32)
    rep = jnp.maximum(s / l, 0.0)
    out_ref[...] = lax.dot_general(
        rep, w_ref[...], (((1,), (1,)), ((), ())),
        preferred_element_type=jnp.float32) + b_ref[...]


def _head(sums, lengths, W, b):
    return pl.pallas_call(
        _head_body,
        out_shape=jax.ShapeDtypeStruct((_B, _OUT), jnp.float32),
    )(sums, lengths, W, b)


def kernel(x, lengths, table, W, b):
    xi = x.astype(jnp.int32)
    li = lengths.astype(jnp.int32)
    tp = jnp.pad(table, ((0, 0), (0, _DP - _D)))
    sums = _embed_sums(xi, li, tp)
    return _head(sums, li.reshape(_B, 1), W, b.reshape(1, _OUT))


# speedup vs baseline: 14.2348x; 1.1255x over previous
import numpy as np
import jax
import jax.numpy as jnp
from jax import lax
from jax.experimental import pallas as pl
from jax.experimental.pallas import tpu as pltpu
from jax.experimental.pallas import tpu_sc as plsc

_B = 4096
_L = 200
_D = 50
_DP = 64
_OUT = 20

_NC = 2
_NS = 16
_NW = _NC * _NS
_SPT = _B // _NW

_CH = 32
_NCH = (_L + _CH - 1) // _CH
_LP = _NCH * _CH

_PERM = np.concatenate([np.arange(0, 32, 2), np.arange(1, 32, 2),
                        np.arange(32, 64, 2), np.arange(33, 64, 2)])
_MASK_HI = np.int32(np.uint32(0xFFFF0000))


def _embed_sums(x, lengths, table):
    mesh = plsc.VectorSubcoreMesh(core_axis_name="c", subcore_axis_name="s")

    def body(x_hbm, len_hbm, table_hbm, sums_hbm, xv, lv, acc, buf0, buf1,
             sem0, sem1):
        wid = lax.axis_index("s") * _NC + lax.axis_index("c")
        base = wid * _SPT
        pltpu.sync_copy(x_hbm.at[pl.ds(base, _SPT)], xv)
        pltpu.sync_copy(len_hbm.at[pl.ds(base, _SPT)], lv.at[pl.ds(0, _SPT)])

        def nchunks(s):
            n = lv[pl.ds(s, 16)][0]
            return (n + (_CH - 1)) // _CH

        def fire(s, buf, sem):
            m = nchunks(s)

            @pl.loop(0, m)
            def _(c):
                pltpu.async_copy(
                    table_hbm.at[xv.at[s, pl.ds(c * _CH, _CH)]],
                    buf.at[pl.ds(c * _CH, _CH)], sem)

        def drain(s, buf, sem):
            m = nchunks(s)

            @pl.loop(0, m)
            def _(c):
                pltpu.make_async_copy(
                    table_hbm.at[xv.at[s, pl.ds(c * _CH, _CH)]],
                    buf.at[pl.ds(c * _CH, _CH)], sem).wait()

        def row_halves(buf, j, col):
            w = plsc.bitcast(buf[j, pl.ds(col, 32)], jnp.int32)
            lo = plsc.bitcast(lax.shift_left(w, 16), jnp.float32)
            hi = plsc.bitcast(lax.bitwise_and(w, _MASK_HI), jnp.float32)
            return lo, hi

        def process(s, buf):
            n = lv[pl.ds(s, 16)][0]
            z = jnp.zeros((16,), jnp.float32)
            n2 = (n // 2) * 2

            def add_row(j, a0, a1, a2, a3):
                lo0, hi0 = row_halves(buf, j, 0)
                lo1, hi1 = row_halves(buf, j, 32)
                return a0 + lo0, a1 + hi0, a2 + lo1, a3 + hi1

            def accum2(j, carry):
                a0, a1, a2, a3, b0, b1, b2, b3 = carry
                a0, a1, a2, a3 = add_row(j, a0, a1, a2, a3)
                b0, b1, b2, b3 = add_row(j + 1, b0, b1, b2, b3)
                return a0, a1, a2, a3, b0, b1, b2, b3

            def accum1(j, carry):
                a0, a1, a2, a3, b0, b1, b2, b3 = carry
                a0, a1, a2, a3 = add_row(j, a0, a1, a2, a3)
                return a0, a1, a2, a3, b0, b1, b2, b3

            carry = (z, z, z, z, z, z, z, z)
            carry = pl.loop(0, n2, step=2, init_carry=carry)(accum2)
            carry = pl.loop(n2, n, init_carry=carry)(accum1)
            a0, a1, a2, a3, b0, b1, b2, b3 = carry
            acc[s, pl.ds(0, 16)] = a0 + b0
            acc[s, pl.ds(16, 16)] = a1 + b1
            acc[s, pl.ds(32, 16)] = a2 + b2
            acc[s, pl.ds(48, 16)] = a3 + b3

        fire(0, buf0, sem0)

        @pl.loop(0, _SPT, step=2)
        def _(s0):
            fire(s0 + 1, buf1, sem1)
            drain(s0, buf0, sem0)
            process(s0, buf0)
            s2 = jnp.where(s0 + 2 >= _SPT, 0, s0 + 2)
            fire(s2, buf0, sem0)
            drain(s0 + 1, buf1, sem1)
            process(s0 + 1, buf1)

        drain(0, buf0, sem0)
        pltpu.sync_copy(acc, sums_hbm.at[pl.ds(base, _SPT)])

    run = pl.kernel(
        body,
        out_type=jax.ShapeDtypeStruct((_B, _DP), jnp.float32),
        mesh=mesh,
        scratch_types=[
            pltpu.VMEM((_SPT, _L), jnp.int32),
            pltpu.VMEM((_SPT + 16,), jnp.int32),
            pltpu.VMEM((_SPT, _DP), jnp.float32),
            pltpu.VMEM((_LP, _DP), jnp.bfloat16),
            pltpu.VMEM((_LP, _DP), jnp.bfloat16),
            pltpu.SemaphoreType.DMA,
            pltpu.SemaphoreType.DMA,
        ],
        compiler_params=pltpu.CompilerParams(use_tc_tiling_on_sc=False,
                                             needs_layout_passes=False),
    )
    return run(x, lengths, table)


def _head_body(sums_ref, len_ref, w_ref, b_ref, out_ref):
    s = sums_ref[...]
    l = len_ref[...].astype(jnp.float32)
    rep = jnp.maximum(s / l, 0.0)
    out_ref[...] = lax.dot_general(
        rep, w_ref[...], (((1,), (1,)), ((), ())),
        preferred_element_type=jnp.float32) + b_ref[...]


def _head(sums, lengths, W, b):
    return pl.pallas_call(
        _head_body,
        out_shape=jax.ShapeDtypeStruct((_B, _OUT), jnp.float32),
    )(sums, lengths, W, b)


def kernel(x, lengths, table, W, b):
    xi = x.astype(jnp.int32)
    li = lengths.astype(jnp.int32)
    tb = jnp.pad(table.astype(jnp.bfloat16), ((0, 0), (0, _DP - _D)))
    sums = _embed_sums(xi, li, tb)
    wp = jnp.pad(W, ((0, 0), (0, _DP - _D)))[:, _PERM]
    return _head(sums, li.reshape(_B, 1), wp, b.reshape(1, _OUT))
